# Initial kernel scaffold; baseline (speedup 1.0000x reference)
import jax
import jax.numpy as jnp
from jax.experimental import pallas as pl


def kernel(tokens, eval, tables, ln_scale, ln_bias):
    raise NotImplementedError("write your pallas kernel here")


# speedup vs baseline: 30.3709x; 30.3709x over previous
import functools

import jax
import jax.numpy as jnp
from jax import lax
from jax.experimental import pallas as pl
from jax.experimental.pallas import tpu as pltpu
from jax.experimental.pallas import tpu_sc as plsc

B = 16384
F = 26
V = 1000
D = 128
L = 16

NC = 2
NS = 16
NW = NC * NS
TPW = B // NW
T = 32
NSUB = TPW // T
KD = D // L
ROWS = T * F
IDX_W = 64
NDMA = ROWS // IDX_W

_mesh = plsc.VectorSubcoreMesh(core_axis_name="c", subcore_axis_name="s")

_GATHER_DNUMS = lax.GatherDimensionNumbers(
    offset_dims=(), collapsed_slice_dims=(0,), start_index_map=(0,))


def _hsum(v, lanes):
    for sh in (8, 4, 2, 1):
        idx = lax.bitwise_xor(lanes, sh).reshape(L, 1)
        v = v + lax.gather(v, idx, _GATHER_DNUMS, slice_sizes=(1,),
                           mode=lax.GatherScatterMode.PROMISE_IN_BOUNDS)
    return v


@functools.partial(
    pl.kernel,
    out_type=jax.ShapeDtypeStruct((B, D), jnp.float32),
    mesh=_mesh,
    scratch_types=[
        pltpu.VMEM((ROWS,), jnp.int32),
        pltpu.VMEM((NDMA, IDX_W), jnp.int32),
        pltpu.VMEM((ROWS, D), jnp.float32),
        pltpu.VMEM((T, D), jnp.float32),
        pltpu.VMEM((D,), jnp.float32),
        pltpu.VMEM((D,), jnp.float32),
        pltpu.SemaphoreType.DMA,
    ],
)
def _emb_ln_kernel(tok_hbm, tab_hbm, scale_hbm, bias_hbm, out_hbm,
                   tok_v, idx_v, rows_v, out_v, scale_v, bias_v, sem):
    wid = lax.axis_index("s") * NC + lax.axis_index("c")
    base = wid * TPW

    pltpu.sync_copy(scale_hbm, scale_v)
    pltpu.sync_copy(bias_hbm, bias_v)

    lanes = lax.iota(jnp.int32, L)

    @pl.loop(0, NSUB)
    def _sub(j):
        t0 = base + j * T

        pltpu.sync_copy(tok_hbm.at[pl.ds(t0 * F, T * F)], tok_v)

        for i in range(ROWS // L):
            tv = tok_v[pl.ds(i * L, L)]
            fld = lax.rem(lanes + i * L, jnp.int32(F))
            g = jnp.minimum(jnp.maximum(tv, 0), V - 1) + fld * V
            row = (i * L) // IDX_W
            col = (i * L) % IDX_W
            idx_v[row, pl.ds(col, L)] = g

        copies = [
            pltpu.async_copy(tab_hbm.at[idx_v.at[n]],
                             rows_v.at[pl.ds(n * IDX_W, IDX_W)], sem)
            for n in range(NDMA)
        ]
        for c in copies:
            c.wait()

        @pl.loop(0, T)
        def _tok(t):
            accs = []
            for k in range(KD):
                a = rows_v[t * F, pl.ds(k * L, L)]
                for f in range(1, F):
                    a = a + rows_v[t * F + f, pl.ds(k * L, L)]
                accs.append(a)
            s = accs[0]
            for k in range(1, KD):
                s = s + accs[k]
            mean = _hsum(s, lanes) * jnp.float32(1.0 / D)
            dif = [a - mean for a in accs]
            vv = dif[0] * dif[0]
            for k in range(1, KD):
                vv = vv + dif[k] * dif[k]
            x = _hsum(vv, lanes) * jnp.float32(1.0 / D) + jnp.float32(1e-12)
            i = lax.bitcast_convert_type(x, jnp.int32)
            i = jnp.int32(0x5F3759DF) - lax.shift_right_logical(i, 1)
            y = lax.bitcast_convert_type(i, jnp.float32)
            half = x * jnp.float32(0.5)
            for _ in range(4):
                y = y * (jnp.float32(1.5) - half * y * y)
            for k in range(KD):
                o = dif[k] * y * scale_v[pl.ds(k * L, L)] + bias_v[pl.ds(k * L, L)]
                out_v[t, pl.ds(k * L, L)] = o

        pltpu.sync_copy(out_v, out_hbm.at[pl.ds(t0, T)])


def kernel(tokens, eval, tables, ln_scale, ln_bias):
    tok_flat = tokens.reshape(-1).astype(jnp.int32)
    tab_flat = tables.reshape(F * V, D)
    return _emb_ln_kernel(tok_flat, tab_flat, ln_scale, ln_bias)


# speedup vs baseline: 66.6594x; 2.1948x over previous
import functools

import jax
import jax.numpy as jnp
from jax import lax
from jax.experimental import pallas as pl
from jax.experimental.pallas import tpu as pltpu
from jax.experimental.pallas import tpu_sc as plsc

B = 16384
F = 26
V = 1000
D = 128
L = 16

NC = 2
NS = 16
NW = NC * NS
TPW = B // NW
T = 128
NSUB = TPW // T
KD = D // L

_mesh = plsc.VectorSubcoreMesh(core_axis_name="c", subcore_axis_name="s")

_GATHER_DNUMS = lax.GatherDimensionNumbers(
    offset_dims=(), collapsed_slice_dims=(0,), start_index_map=(0,))


def _hsum(v, lanes):
    for sh in (8, 4, 2, 1):
        idx = lax.bitwise_xor(lanes, sh).reshape(L, 1)
        v = v + lax.gather(v, idx, _GATHER_DNUMS, slice_sizes=(1,),
                           mode=lax.GatherScatterMode.PROMISE_IN_BOUNDS)
    return v


@functools.partial(
    pl.kernel,
    out_type=jax.ShapeDtypeStruct((B, D), jnp.float32),
    mesh=_mesh,
    scratch_types=[
        pltpu.VMEM((F, T), jnp.int32),
        pltpu.VMEM((F, T), jnp.int32),
        pltpu.VMEM((T, D), jnp.float32),
        pltpu.VMEM((T, D), jnp.float32),
        pltpu.VMEM((T, D), jnp.float32),
        pltpu.VMEM((T, D), jnp.float32),
        pltpu.VMEM((D,), jnp.float32),
        pltpu.VMEM((D,), jnp.float32),
        pltpu.SemaphoreType.DMA,
        pltpu.SemaphoreType.DMA,
        pltpu.SemaphoreType.DMA,
        pltpu.SemaphoreType.DMA,
    ],
)
def _emb_ln_kernel(tokT_hbm, tab_hbm, scale_hbm, bias_hbm, out_hbm,
                   idx0, idx1, acc0, acc1, ob0, ob1,
                   scale_v, bias_v, g0, g1, o0, o1):
    wid = lax.axis_index("s") * NC + lax.axis_index("c")
    base = wid * TPW

    pltpu.sync_copy(scale_hbm, scale_v)
    pltpu.sync_copy(bias_hbm, bias_v)

    idx_b = (idx0, idx1)
    acc_b = (acc0, acc1)
    ob_b = (ob0, ob1)
    g_b = (g0, g1)
    o_b = (o0, o1)

    def stage(j):
        p = j % 2
        idx_v, acc, gsem = idx_b[p], acc_b[p], g_b[p]
        t0 = base + j * T
        pltpu.sync_copy(tokT_hbm.at[:, pl.ds(t0, T)], idx_v)

        @pl.loop(0, T // L)
        def _idx(tb):
            for f in range(F):
                v = idx_v[f, pl.ds(tb * L, L)]
                idx_v[f, pl.ds(tb * L, L)] = (
                    jnp.minimum(jnp.maximum(v, 0), V - 1) + f * V)

        zeros = jnp.zeros((L,), jnp.float32)

        @pl.loop(0, T)
        def _zero(r):
            for k in range(KD):
                acc[r, pl.ds(k * L, L)] = zeros

        return [pltpu.async_copy(tab_hbm.at[idx_v.at[f]], acc, gsem,
                                 add=True)
                for f in range(F)]

    def finish(j):
        p = j % 2
        acc, ob, osem = acc_b[p], ob_b[p], o_b[p]
        t0 = base + j * T
        lanes = lax.iota(jnp.int32, L)

        @pl.loop(0, T)
        def _tok(t):
            accs = [acc[t, pl.ds(k * L, L)] for k in range(KD)]
            s = accs[0]
            for k in range(1, KD):
                s = s + accs[k]
            mean = _hsum(s, lanes) * jnp.float32(1.0 / D)
            dif = [a - mean for a in accs]
            vv = dif[0] * dif[0]
            for k in range(1, KD):
                vv = vv + dif[k] * dif[k]
            x = _hsum(vv, lanes) * jnp.float32(1.0 / D) + jnp.float32(1e-12)
            i = lax.bitcast_convert_type(x, jnp.int32)
            i = jnp.int32(0x5F3759DF) - lax.shift_right_logical(i, 1)
            y = lax.bitcast_convert_type(i, jnp.float32)
            half = x * jnp.float32(0.5)
            for _ in range(4):
                y = y * (jnp.float32(1.5) - half * y * y)
            for k in range(KD):
                o = (dif[k] * y * scale_v[pl.ds(k * L, L)]
                     + bias_v[pl.ds(k * L, L)])
                ob[t, pl.ds(k * L, L)] = o

        return pltpu.async_copy(ob, out_hbm.at[pl.ds(t0, T)], osem)

    gather_h = {0: stage(0)}
    out_h = {}
    for j in range(NSUB):
        if j + 1 < NSUB:
            gather_h[j + 1] = stage(j + 1)
        for c in gather_h.pop(j):
            c.wait()
        if j - 2 in out_h:
            out_h.pop(j - 2).wait()
        out_h[j] = finish(j)
    for j in sorted(out_h):
        out_h.pop(j).wait()


def kernel(tokens, eval, tables, ln_scale, ln_bias):
    tok_t = tokens.astype(jnp.int32).T
    tab_flat = tables.reshape(F * V, D)
    return _emb_ln_kernel(tok_t, tab_flat, ln_scale, ln_bias)
